# Initial kernel scaffold; baseline (speedup 1.0000x reference)
import jax
import jax.numpy as jnp
from jax.experimental import pallas as pl


def kernel(node_embeddings, batch, W_gate, b_gate, W_t, b_t, W1, b1, W2, b2):
    raise NotImplementedError("write your pallas kernel here")


# speedup vs baseline: 2.2643x; 2.2643x over previous
import functools

import jax
import jax.numpy as jnp
from jax import lax
from jax.experimental import pallas as pl
from jax.experimental.pallas import tpu as pltpu
from jax.experimental.pallas import tpu_sc as plsc

N_NODES = 100000
D = 128
G = 64
NUM_CLASSES = 10
NW = 32
CR = 224

_BIG = 196 * 16
_SMALL = 195 * 16


def _sc_gated_segment_sum(x, batch, wg, bg16):
  mesh = plsc.VectorSubcoreMesh(core_axis_name="c", subcore_axis_name="s")

  @functools.partial(
      pl.kernel,
      mesh=mesh,
      out_type=(
          jax.ShapeDtypeStruct((NW, G, D), jnp.float32),
          jax.ShapeDtypeStruct((NW, G, 16), jnp.float32),
      ),
      scratch_types=[
          pltpu.VMEM((CR, D), jnp.float32),
          pltpu.VMEM((CR,), jnp.int32),
          pltpu.VMEM((G, D), jnp.float32),
          pltpu.VMEM((G, 16), jnp.float32),
          pltpu.VMEM((D,), jnp.float32),
          pltpu.VMEM((16,), jnp.float32),
      ],
  )
  def k(x_hbm, b_hbm, wg_hbm, bg_hbm, out_s, out_c, xbuf, bbuf, acc, accc,
        wgbuf, bgbuf):
    cid = lax.axis_index("c")
    sid = lax.axis_index("s")
    wid = sid * 2 + cid

    start = jnp.minimum(wid, 10) * _BIG + jnp.maximum(wid - 10, 0) * _SMALL
    rows = jnp.where(wid < 10, _BIG, _SMALL)

    pltpu.sync_copy(wg_hbm, wgbuf)
    pltpu.sync_copy(bg_hbm, bgbuf)

    zero16 = jnp.zeros((16,), jnp.float32)

    def zero_body(g, _):
      for j in range(8):
        acc[g, pl.ds(16 * j, 16)] = zero16
      accc[g, :] = zero16
      return 0

    lax.fori_loop(0, G, zero_body, 0)

    nch = (rows + CR - 1) // CR

    def chunk_body(kc, _):
      bk = jnp.minimum(kc * CR, rows - CR)
      st = kc * CR - bk
      pltpu.sync_copy(x_hbm.at[pl.ds(start + bk, CR), :], xbuf)
      pltpu.sync_copy(b_hbm.at[pl.ds(start + bk, CR)], bbuf)

      wgv = [wgbuf[pl.ds(16 * j, 16)] for j in range(8)]
      bgv = bgbuf[...]

      def group_body(gi, _):
        i0 = gi * 16
        bvec = bbuf[pl.ds(i0, 16)]
        for r in range(16):
          i = i0 + r
          b = bvec[r]
          xv = [xbuf[i, pl.ds(16 * j, 16)] for j in range(8)]
          zv = ((xv[0] * wgv[0] + xv[1] * wgv[1])
                + (xv[2] * wgv[2] + xv[3] * wgv[3])) + \
               ((xv[4] * wgv[4] + xv[5] * wgv[5])
                + (xv[6] * wgv[6] + xv[7] * wgv[7]))
          p = zv
          for s in (8, 4, 2, 1):
            idx = lax.iota(jnp.int32, 16) ^ s
            p = p + p.at[idx].get(mode="promise_in_bounds")
          gv = 1.0 / (1.0 + jnp.exp(-(p + bgv)))
          for j in range(8):
            plsc.addupdate(acc.at[b, pl.ds(16 * j, 16)], xv[j] * gv)
          plsc.addupdate(accc.at[b, :], gv)
        return 0

      lax.fori_loop(st // 16, CR // 16, group_body, 0)
      return 0

    lax.fori_loop(0, nch, chunk_body, 0)

    pltpu.sync_copy(acc, out_s.at[wid])
    pltpu.sync_copy(accc, out_c.at[wid])

  return k(x, batch, wg, bg16)


def _tc_head(s_all, c_all, w_t, b_t, w1, b1, w2, b2):
  def body(s_ref, c_ref, wt_ref, bt_ref, w1_ref, b1_ref, w2_ref, b2_ref,
           o_ref):
    s = jnp.sum(s_ref[...], axis=0)
    c = jnp.sum(c_ref[...], axis=(0, 2))[:, None] * (1.0 / 16.0)
    hg = jnp.dot(s, wt_ref[...], preferred_element_type=jnp.float32) \
        + c * bt_ref[...]
    pre = jnp.dot(hg, w1_ref[...], preferred_element_type=jnp.float32) \
        + b1_ref[...]
    hid = 0.5 * pre * (1.0 + lax.erf(pre * (2.0 ** -0.5)))
    o_ref[...] = jnp.dot(hid, w2_ref[...],
                         preferred_element_type=jnp.float32) + b2_ref[...]

  return pl.pallas_call(
      body,
      out_shape=jax.ShapeDtypeStruct((G, NUM_CLASSES), jnp.float32),
  )(s_all, c_all, w_t, b_t, w1, b1, w2, b2)


def kernel(node_embeddings, batch, W_gate, b_gate, W_t, b_t, W1, b1, W2, b2):
  x = node_embeddings.astype(jnp.float32)
  bi = batch.astype(jnp.int32)
  wg = W_gate.reshape(D)
  bg16 = jnp.broadcast_to(b_gate.reshape(1), (16,)).astype(jnp.float32)
  s_all, c_all = _sc_gated_segment_sum(x, bi, wg, bg16)
  return _tc_head(s_all, c_all, W_t, b_t.reshape(1, D), W1, b1.reshape(1, D),
                  W2, b2.reshape(1, NUM_CLASSES))


# speedup vs baseline: 2.2927x; 1.0125x over previous
import functools

import jax
import jax.numpy as jnp
from jax import lax
from jax.experimental import pallas as pl
from jax.experimental.pallas import tpu as pltpu
from jax.experimental.pallas import tpu_sc as plsc

N_NODES = 100000
D = 128
G = 64
NUM_CLASSES = 10
NW = 32
CR = 224

_BIG = 196 * 16
_SMALL = 195 * 16


def _sc_gated_segment_sum(x, batch, wg, bg16):
  mesh = plsc.VectorSubcoreMesh(core_axis_name="c", subcore_axis_name="s")

  @functools.partial(
      pl.kernel,
      mesh=mesh,
      out_type=(
          jax.ShapeDtypeStruct((NW, G, D), jnp.float32),
          jax.ShapeDtypeStruct((NW, G, 16), jnp.float32),
      ),
      scratch_types=[
          pltpu.VMEM((CR, D), jnp.float32),
          pltpu.VMEM((CR, D), jnp.float32),
          pltpu.VMEM((CR,), jnp.int32),
          pltpu.VMEM((CR,), jnp.int32),
          pltpu.VMEM((G, D), jnp.float32),
          pltpu.VMEM((G, 16), jnp.float32),
          pltpu.VMEM((D,), jnp.float32),
          pltpu.VMEM((16,), jnp.float32),
          pltpu.SemaphoreType.DMA,
          pltpu.SemaphoreType.DMA,
          pltpu.SemaphoreType.DMA,
          pltpu.SemaphoreType.DMA,
      ],
  )
  def k(x_hbm, b_hbm, wg_hbm, bg_hbm, out_s, out_c, xbuf0, xbuf1, bbuf0,
        bbuf1, acc, accc, wgbuf, bgbuf, sx0, sx1, sb0, sb1):
    cid = lax.axis_index("c")
    sid = lax.axis_index("s")
    wid = sid * 2 + cid

    start = jnp.minimum(wid, 10) * _BIG + jnp.maximum(wid - 10, 0) * _SMALL
    rows = jnp.where(wid < 10, _BIG, _SMALL)

    pltpu.sync_copy(wg_hbm, wgbuf)
    pltpu.sync_copy(bg_hbm, bgbuf)

    zero16 = jnp.zeros((16,), jnp.float32)

    @plsc.parallel_loop(0, G)
    def _zero(g):
      for j in range(8):
        acc[g, pl.ds(16 * j, 16)] = zero16
      accc[g, :] = zero16

    NCH = 14
    assert NCH * CR >= _BIG and (NCH - 1) * CR < _SMALL

    def chunk_base(kc):
      return jnp.minimum(kc * CR, rows - CR)

    def xsl(kc):
      return x_hbm.at[pl.ds(start + chunk_base(kc), CR), :]

    def bsl(kc):
      return b_hbm.at[pl.ds(start + chunk_base(kc), CR)]

    def start_pair(kc, xbuf, bbuf, sx, sb):
      pltpu.async_copy(xsl(kc), xbuf, sx)
      pltpu.async_copy(bsl(kc), bbuf, sb)

    wgv = [wgbuf[pl.ds(16 * j, 16)] for j in range(8)]
    bgv = bgbuf[...]

    def process(kc, xbuf, bbuf):
      g_lo = (kc * CR - chunk_base(kc)) // 16

      @plsc.parallel_loop(g_lo, CR // 16, unroll=2)
      def _group(gi):
        i0 = gi * 16
        bvec = bbuf[pl.ds(i0, 16)]
        for r in range(16):
          i = i0 + r
          b = bvec[r]
          xv = [xbuf[i, pl.ds(16 * j, 16)] for j in range(8)]
          zv = ((xv[0] * wgv[0] + xv[1] * wgv[1])
                + (xv[2] * wgv[2] + xv[3] * wgv[3])) + \
               ((xv[4] * wgv[4] + xv[5] * wgv[5])
                + (xv[6] * wgv[6] + xv[7] * wgv[7]))
          p = zv
          for s in (8, 4, 2, 1):
            idx = lax.iota(jnp.int32, 16) ^ s
            p = p + p.at[idx].get(mode="promise_in_bounds")
          gv = 1.0 / (1.0 + jnp.exp(-(p + bgv)))
          for j in range(8):
            plsc.addupdate(acc.at[b, pl.ds(16 * j, 16)], xv[j] * gv)
          plsc.addupdate(accc.at[b, :], gv)

    start_pair(0, xbuf0, bbuf0, sx0, sb0)
    start_pair(1, xbuf1, bbuf1, sx1, sb1)

    def pair_body(t, _):
      k0 = 2 * t
      k1 = k0 + 1
      pltpu.make_async_copy(xsl(k0), xbuf0, sx0).wait()
      pltpu.make_async_copy(bsl(k0), bbuf0, sb0).wait()
      process(k0, xbuf0, bbuf0)

      @pl.when(k0 + 2 < NCH)
      def _():
        start_pair(k0 + 2, xbuf0, bbuf0, sx0, sb0)

      pltpu.make_async_copy(xsl(k1), xbuf1, sx1).wait()
      pltpu.make_async_copy(bsl(k1), bbuf1, sb1).wait()
      process(k1, xbuf1, bbuf1)

      @pl.when(k1 + 2 < NCH)
      def _():
        start_pair(k1 + 2, xbuf1, bbuf1, sx1, sb1)

      return 0

    lax.fori_loop(0, NCH // 2, pair_body, 0)

    pltpu.sync_copy(acc, out_s.at[wid])
    pltpu.sync_copy(accc, out_c.at[wid])

  return k(x, batch, wg, bg16)


def _tc_head(s_all, c_all, w_t, b_t, w1, b1, w2, b2):
  def body(s_ref, c_ref, wt_ref, bt_ref, w1_ref, b1_ref, w2_ref, b2_ref,
           o_ref):
    s = jnp.sum(s_ref[...], axis=0)
    c = jnp.sum(c_ref[...], axis=(0, 2))[:, None] * (1.0 / 16.0)
    hg = jnp.dot(s, wt_ref[...], preferred_element_type=jnp.float32) \
        + c * bt_ref[...]
    pre = jnp.dot(hg, w1_ref[...], preferred_element_type=jnp.float32) \
        + b1_ref[...]
    hid = 0.5 * pre * (1.0 + lax.erf(pre * (2.0 ** -0.5)))
    o_ref[...] = jnp.dot(hid, w2_ref[...],
                         preferred_element_type=jnp.float32) + b2_ref[...]

  return pl.pallas_call(
      body,
      out_shape=jax.ShapeDtypeStruct((G, NUM_CLASSES), jnp.float32),
  )(s_all, c_all, w_t, b_t, w1, b1, w2, b2)


def kernel(node_embeddings, batch, W_gate, b_gate, W_t, b_t, W1, b1, W2, b2):
  x = node_embeddings.astype(jnp.float32)
  bi = batch.astype(jnp.int32)
  wg = W_gate.reshape(D)
  bg16 = jnp.broadcast_to(b_gate.reshape(1), (16,)).astype(jnp.float32)
  s_all, c_all = _sc_gated_segment_sum(x, bi, wg, bg16)
  return _tc_head(s_all, c_all, W_t, b_t.reshape(1, D), W1, b1.reshape(1, D),
                  W2, b2.reshape(1, NUM_CLASSES))
